# baseline (speedup 1.0000x reference)
import functools

import jax
import jax.numpy as jnp
import numpy as np
from jax import lax
from jax.experimental import pallas as pl
from jax.experimental.pallas import tpu as pltpu
from jax.experimental.pallas import tpu_sc as plsc

N = 8192
NC, NS, L = 2, 16, 16
NW = NC * NS
QPW = N // NW
CPG = 4
SUP = L * CPG
NSUP = N // SUP
SPW = NSUP // NW
NSEG = 8
BZ = 16
NBIN = NSEG * BZ

R1 = np.float32(0.2)
R2 = np.float32(0.4)
T1 = np.float32(0.2 * 0.2)
T2 = np.float32(0.4 * 0.4)


def _sc_counts(xs, ys, zs, b32):
    mesh = plsc.VectorSubcoreMesh(
        core_axis_name="c", subcore_axis_name="s",
        num_cores=NC, num_subcores=NS)

    @functools.partial(
        pl.kernel,
        out_type=(jax.ShapeDtypeStruct((N,), jnp.float32),
                  jax.ShapeDtypeStruct((N,), jnp.float32),
                  jax.ShapeDtypeStruct((N,), jnp.int32)),
        mesh=mesh,
        scratch_types=[
            pltpu.VMEM((N,), jnp.float32),
            pltpu.VMEM((N,), jnp.float32),
            pltpu.VMEM((N,), jnp.float32),
            pltpu.VMEM((N,), jnp.int32),
            pltpu.VMEM((N,), jnp.float32),
            pltpu.VMEM((N,), jnp.float32),
            pltpu.VMEM((N,), jnp.float32),
            pltpu.VMEM((N,), jnp.float32),
            pltpu.VMEM((N,), jnp.int32),
            pltpu.VMEM((N,), jnp.int32),
            pltpu.VMEM(((NBIN + 1) * L,), jnp.int32),
            pltpu.VMEM(((NBIN + 1) * L,), jnp.int32),
            pltpu.VMEM((NBIN * L,), jnp.int32),
            pltpu.VMEM((SUP,), jnp.float32),
            pltpu.VMEM((SUP,), jnp.float32),
            pltpu.SemaphoreType.DMA,
        ],
        compiler_params=pltpu.CompilerParams(
            use_tc_tiling_on_sc=False, needs_layout_passes=False),
    )
    def k(xs_h, ys_h, zs_h, b_h, c1_h, c2_h, inv_h,
          xs_v, ys_v, zs_v, b_v, x2_v, y2_v, z2_v, hw2_v, b2_v, inv_v,
          hist_v, ls_v, cur_v, c1s_v, c2s_v, dsem):
        wid = lax.axis_index("s") * NC + lax.axis_index("c")
        cps = [pltpu.async_copy(s, d, dsem)
               for s, d in ((xs_h, xs_v), (ys_h, ys_v), (zs_h, zs_v),
                            (b_h, b_v))]
        for cp in cps:
            cp.wait()
        lane = lax.iota(jnp.int32, L)
        zeros_i = jnp.zeros((L,), jnp.int32)
        ones_i = jnp.full((L,), 1, jnp.int32)

        def zero_body(i, _):
            plsc.store_scatter(hist_v, [i * L + lane], zeros_i)
            return 0
        lax.fori_loop(0, NBIN + 1, zero_body, 0, unroll=4)

        def bin_of(zv, bv):
            zb = jnp.clip((zv * np.float32(BZ)).astype(jnp.int32),
                          0, BZ - 1)
            return bv * BZ + zb

        def hist_body(i, _):
            idx = i * L + lane
            zv = plsc.load_gather(zs_v, [idx])
            bv = plsc.load_gather(b_v, [idx])
            gl = bin_of(zv, bv) * L + lane
            plsc.addupdate_scatter(hist_v, [gl], ones_i)
            return 0
        lax.fori_loop(0, N // L, hist_body, 0, unroll=4)

        def pfx_body(g, base):
            gl = g * L + lane
            cnt = plsc.load_gather(hist_v, [gl])
            ex = plsc.cumsum(cnt) - cnt + base
            plsc.store_scatter(ls_v, [gl], ex)
            plsc.store_scatter(cur_v, [gl], ex)
            return base + jnp.sum(cnt)
        total = lax.fori_loop(0, NBIN, pfx_body, jnp.int32(0))
        plsc.store_scatter(ls_v, [NBIN * L + lane],
                           jnp.full((L,), 1, jnp.int32) * total)

        def perm_body(i, _):
            idx = i * L + lane
            xv = plsc.load_gather(xs_v, [idx])
            yv = plsc.load_gather(ys_v, [idx])
            zv = plsc.load_gather(zs_v, [idx])
            bv = plsc.load_gather(b_v, [idx])
            gl = bin_of(zv, bv) * L + lane
            p = plsc.load_gather(cur_v, [gl])
            plsc.store_scatter(cur_v, [gl], p + ones_i)
            plsc.store_scatter(x2_v, [p], xv)
            plsc.store_scatter(y2_v, [p], yv)
            plsc.store_scatter(z2_v, [p], zv)
            plsc.store_scatter(
                hw2_v, [p], (xv * xv + yv * yv + zv * zv) * np.float32(0.5))
            plsc.store_scatter(b2_v, [p], bv)
            plsc.store_scatter(inv_v, [idx], p)
            return 0
        lax.fori_loop(0, N // L, perm_body, 0, unroll=2)

        def sup_body(t, _):
            qb = (t * NW + wid) * SUP
            qx, qy, qz, bq, ht1, ht2 = [], [], [], [], [], []
            jlo2 = jnp.int32(N)
            jlo1 = jnp.int32(N)
            jhi1 = jnp.int32(0)
            jhi2 = jnp.int32(0)
            bmin = jnp.int32(127)
            bmax = jnp.int32(-1)
            for g in range(CPG):
                qidx = qb + g * L + lane
                qx.append(plsc.load_gather(x2_v, [qidx]))
                qy.append(plsc.load_gather(y2_v, [qidx]))
                qz.append(plsc.load_gather(z2_v, [qidx]))
                bq.append(plsc.load_gather(b2_v, [qidx]))
                qn = qx[g] * qx[g] + qy[g] * qy[g] + qz[g] * qz[g]
                ht1.append((T1 - qn) * np.float32(0.5))
                ht2.append((T2 - qn) * np.float32(0.5))
                glo1 = bin_of(qz[g] - R1, bq[g])
                ghi1 = bin_of(qz[g] + R1, bq[g])
                glo2 = bin_of(qz[g] - R2, bq[g])
                ghi2 = bin_of(qz[g] + R2, bq[g])
                lo1 = plsc.load_gather(ls_v, [glo1 * L])
                hi1 = plsc.load_gather(ls_v, [(ghi1 + 1) * L])
                lo2 = plsc.load_gather(ls_v, [glo2 * L])
                hi2 = plsc.load_gather(ls_v, [(ghi2 + 1) * L])
                jlo1 = jnp.minimum(jlo1, jnp.min(lo1))
                jhi1 = jnp.maximum(jhi1, jnp.max(hi1))
                jlo2 = jnp.minimum(jlo2, jnp.min(lo2))
                jhi2 = jnp.maximum(jhi2, jnp.max(hi2))
                bmin = jnp.minimum(bmin, jnp.min(bq[g]))
                bmax = jnp.maximum(bmax, jnp.max(bq[g]))
            jlo1 = jnp.maximum(jlo1, jlo2)
            jhi1 = jnp.minimum(jhi1, jhi2)
            jlo1 = jnp.minimum(jlo1, jhi1)

            both = jnp.int32(0x10001)
            one_i = jnp.int32(1)
            zero_i = jnp.int32(0)
            z16 = jnp.zeros((L,), jnp.int32)

            def make_body(masked, small):
                def body(j, carry):
                    jv = jnp.full((L,), j, dtype=jnp.int32)
                    xj = plsc.load_gather(x2_v, [jv])
                    yj = plsc.load_gather(y2_v, [jv])
                    zj = plsc.load_gather(z2_v, [jv])
                    hwj = plsc.load_gather(hw2_v, [jv])
                    bj = plsc.load_gather(b2_v, [jv]) if masked else None
                    out = []
                    for g in range(CPG):
                        u = hwj - (qx[g] * xj + qy[g] * yj + qz[g] * zj)
                        m2 = u <= ht2[g]
                        if masked:
                            m2 = m2 & (bj == bq[g])
                        if small:
                            m1 = u <= ht1[g]
                            step = jnp.where(
                                m2, jnp.where(m1, both, one_i), zero_i)
                        else:
                            step = jnp.where(m2, one_i, zero_i)
                        out.append(carry[g] + step)
                    return tuple(out)
                return body

            def run(masked):
                def f(_):
                    a = plsc.parallel_loop(
                        jlo2, jlo1, 1, unroll=2,
                        carry=(z16,) * CPG)(make_body(masked, False))
                    a = plsc.parallel_loop(
                        jlo1, jhi1, 1, unroll=2,
                        carry=a)(make_body(masked, True))
                    a = plsc.parallel_loop(
                        jhi1, jhi2, 1, unroll=2,
                        carry=a)(make_body(masked, False))
                    return a
                return f

            acc = lax.cond(bmin == bmax, run(False), run(True), 0)
            for g in range(CPG):
                c1 = (acc[g] >> 16).astype(jnp.float32)
                c2 = (acc[g] & jnp.int32(0xFFFF)).astype(jnp.float32)
                c1s_v[pl.ds(g * L, L)] = (
                    jnp.minimum(c1, np.float32(32.0)) * np.float32(1.0 / 32.0))
                c2s_v[pl.ds(g * L, L)] = (
                    jnp.minimum(c2, np.float32(64.0)) * np.float32(1.0 / 64.0))
            pltpu.sync_copy(c1s_v, c1_h.at[pl.ds(qb, SUP)])
            pltpu.sync_copy(c2s_v, c2_h.at[pl.ds(qb, SUP)])
            return 0
        lax.fori_loop(0, SPW, sup_body, 0)

        ob = wid * QPW
        pltpu.sync_copy(inv_v.at[pl.ds(ob, QPW)], inv_h.at[pl.ds(ob, QPW)])

    return k(xs, ys, zs, b32)


def kernel(x, pos, batch):
    pos = pos.astype(jnp.float32)
    xs = pos[:, 0]
    ys = pos[:, 1]
    zs = pos[:, 2]
    b32 = batch.astype(jnp.int32)
    c1p, c2p, inv = _sc_counts(xs, ys, zs, b32)
    c1 = c1p[inv]
    c2 = c2p[inv]
    feats = jnp.concatenate([x, pos, c1[:, None], c2[:, None]], axis=1)
    return feats, pos, batch


# speedup vs baseline: 1.1626x; 1.1626x over previous
import functools

import jax
import jax.numpy as jnp
import numpy as np
from jax import lax
from jax.experimental import pallas as pl
from jax.experimental.pallas import tpu as pltpu
from jax.experimental.pallas import tpu_sc as plsc

N = 8192
NC, NS, L = 2, 16, 16
NW = NC * NS
QPW = N // NW
CHUNKS = QPW // L
CPG = 4

T1 = np.float32(0.2 * 0.2)
T2 = np.float32(0.4 * 0.4)


def _sc_counts(xs, ys, zs, b32, off16):
    mesh = plsc.VectorSubcoreMesh(
        core_axis_name="c", subcore_axis_name="s",
        num_cores=NC, num_subcores=NS)

    @functools.partial(
        pl.kernel,
        out_type=(jax.ShapeDtypeStruct((N,), jnp.float32),
                  jax.ShapeDtypeStruct((N,), jnp.float32)),
        mesh=mesh,
        scratch_types=[
            pltpu.VMEM((N,), jnp.float32),
            pltpu.VMEM((N,), jnp.float32),
            pltpu.VMEM((N,), jnp.float32),
            pltpu.VMEM((N,), jnp.int32),
            pltpu.VMEM((L,), jnp.int32),
            pltpu.VMEM((N,), jnp.float32),
            pltpu.VMEM((QPW,), jnp.float32),
            pltpu.VMEM((QPW,), jnp.float32),
            pltpu.SemaphoreType.DMA,
        ],
        compiler_params=pltpu.CompilerParams(
            use_tc_tiling_on_sc=False, needs_layout_passes=False),
    )
    def k(xs_h, ys_h, zs_h, b_h, off_h, c1_h, c2_h,
          xs_v, ys_v, zs_v, b_v, off_v, hw_v, c1_v, c2_v, dsem):
        wid = lax.axis_index("s") * NC + lax.axis_index("c")
        cps = [pltpu.async_copy(s, d, dsem)
               for s, d in ((xs_h, xs_v), (ys_h, ys_v), (zs_h, zs_v),
                            (b_h, b_v), (off_h, off_v))]
        for cp in cps:
            cp.wait()
        qbase = wid * QPW
        lane = lax.iota(jnp.int32, L)

        def hw_body(i, _):
            idx = i * L + lane
            xv = plsc.load_gather(xs_v, [idx])
            yv = plsc.load_gather(ys_v, [idx])
            zv = plsc.load_gather(zs_v, [idx])
            plsc.store_scatter(
                hw_v, [idx],
                (xv * xv + yv * yv + zv * zv) * np.float32(0.5))
            return 0
        lax.fori_loop(0, N // L, hw_body, 0, unroll=4)

        for sc in range(QPW // (L * CPG)):
            qx, qy, qz, bq, ht1, ht2 = [], [], [], [], [], []
            jstart = jnp.int32(N)
            jend = jnp.int32(0)
            bmin = jnp.int32(127)
            bmax = jnp.int32(-1)
            for g in range(CPG):
                qidx = qbase + (sc * CPG + g) * L + lane
                qx.append(plsc.load_gather(xs_v, [qidx]))
                qy.append(plsc.load_gather(ys_v, [qidx]))
                qz.append(plsc.load_gather(zs_v, [qidx]))
                bq.append(plsc.load_gather(b_v, [qidx]))
                qn = qx[g] * qx[g] + qy[g] * qy[g] + qz[g] * qz[g]
                ht1.append((T1 - qn) * np.float32(0.5))
                ht2.append((T2 - qn) * np.float32(0.5))
                sv = plsc.load_gather(off_v, [bq[g]])
                ev = plsc.load_gather(off_v, [bq[g] + 1])
                jstart = jnp.minimum(jstart, jnp.min(sv))
                jend = jnp.maximum(jend, jnp.max(ev))
                bmin = jnp.minimum(bmin, jnp.min(bq[g]))
                bmax = jnp.maximum(bmax, jnp.max(bq[g]))

            both = jnp.int32(0x10001)
            one_i = jnp.int32(1)
            zero_i = jnp.int32(0)
            z16 = jnp.zeros((L,), jnp.int32)

            def make_body(masked):
                def body(j, carry):
                    jv = jnp.full((L,), j, dtype=jnp.int32)
                    xj = plsc.load_gather(xs_v, [jv])
                    yj = plsc.load_gather(ys_v, [jv])
                    zj = plsc.load_gather(zs_v, [jv])
                    hwj = plsc.load_gather(hw_v, [jv])
                    bj = plsc.load_gather(b_v, [jv]) if masked else None
                    out = []
                    for g in range(CPG):
                        u = hwj - (qx[g] * xj + qy[g] * yj + qz[g] * zj)
                        m1 = u <= ht1[g]
                        m2 = u <= ht2[g]
                        if masked:
                            m2 = m2 & (bj == bq[g])
                        step = jnp.where(m2, jnp.where(m1, both, one_i), zero_i)
                        out.append(carry[g] + step)
                    return tuple(out)
                return body

            def run(masked):
                def f(_):
                    return plsc.parallel_loop(
                        jstart, jend, 1, unroll=2,
                        carry=(z16,) * CPG)(make_body(masked))
                return f

            acc = lax.cond(bmin == bmax, run(False), run(True), 0)
            for g in range(CPG):
                o = (sc * CPG + g) * L
                c1 = (acc[g] >> 16).astype(jnp.float32)
                c2 = (acc[g] & jnp.int32(0xFFFF)).astype(jnp.float32)
                c1_v[pl.ds(o, L)] = (
                    jnp.minimum(c1, np.float32(32.0)) * np.float32(1.0 / 32.0))
                c2_v[pl.ds(o, L)] = (
                    jnp.minimum(c2, np.float32(64.0)) * np.float32(1.0 / 64.0))

        pltpu.sync_copy(c1_v, c1_h.at[pl.ds(qbase, QPW)])
        pltpu.sync_copy(c2_v, c2_h.at[pl.ds(qbase, QPW)])

    return k(xs, ys, zs, b32, off16)


def kernel(x, pos, batch):
    pos = pos.astype(jnp.float32)
    xs = pos[:, 0]
    ys = pos[:, 1]
    zs = pos[:, 2]
    b32 = batch.astype(jnp.int32)
    off16 = jnp.sum(b32[:, None] < jnp.arange(L, dtype=jnp.int32)[None, :],
                    axis=0, dtype=jnp.int32)
    c1, c2 = _sc_counts(xs, ys, zs, b32, off16)
    feats = jnp.concatenate([x, pos, c1[:, None], c2[:, None]], axis=1)
    return feats, pos, batch


# speedup vs baseline: 1.1671x; 1.0038x over previous
import functools

import jax
import jax.numpy as jnp
import numpy as np
from jax import lax
from jax.experimental import pallas as pl
from jax.experimental.pallas import tpu as pltpu
from jax.experimental.pallas import tpu_sc as plsc

N = 8192
NC, NS, L = 2, 16, 16
NW = NC * NS
QPW = N // NW
CHUNKS = QPW // L
CPG = 4

T1 = np.float32(0.2 * 0.2)
T2 = np.float32(0.4 * 0.4)


def _sc_counts(xs, ys, zs, b32, off16):
    mesh = plsc.VectorSubcoreMesh(
        core_axis_name="c", subcore_axis_name="s",
        num_cores=NC, num_subcores=NS)

    @functools.partial(
        pl.kernel,
        out_type=(jax.ShapeDtypeStruct((N,), jnp.float32),
                  jax.ShapeDtypeStruct((N,), jnp.float32)),
        mesh=mesh,
        scratch_types=[
            pltpu.VMEM((N,), jnp.float32),
            pltpu.VMEM((N,), jnp.float32),
            pltpu.VMEM((N,), jnp.float32),
            pltpu.VMEM((N,), jnp.int32),
            pltpu.VMEM((L,), jnp.int32),
            pltpu.VMEM((N,), jnp.float32),
            pltpu.VMEM((QPW,), jnp.float32),
            pltpu.VMEM((QPW,), jnp.float32),
            pltpu.SemaphoreType.DMA,
        ],
        compiler_params=pltpu.CompilerParams(
            use_tc_tiling_on_sc=False, needs_layout_passes=False),
    )
    def k(xs_h, ys_h, zs_h, b_h, off_h, c1_h, c2_h,
          xs_v, ys_v, zs_v, b_v, off_v, hw_v, c1_v, c2_v, dsem):
        wid = lax.axis_index("s") * NC + lax.axis_index("c")
        cps = [pltpu.async_copy(s, d, dsem)
               for s, d in ((xs_h, xs_v), (ys_h, ys_v), (zs_h, zs_v),
                            (b_h, b_v), (off_h, off_v))]
        for cp in cps:
            cp.wait()
        qbase = wid * QPW
        lane = lax.iota(jnp.int32, L)

        def hw_body(i, _):
            idx = i * L + lane
            xv = plsc.load_gather(xs_v, [idx])
            yv = plsc.load_gather(ys_v, [idx])
            zv = plsc.load_gather(zs_v, [idx])
            plsc.store_scatter(
                hw_v, [idx],
                (xv * xv + yv * yv + zv * zv) * np.float32(0.5))
            return 0
        lax.fori_loop(0, N // L, hw_body, 0, unroll=4)

        for sc in range(QPW // (L * CPG)):
            qx, qy, qz, bq, ht1, ht2 = [], [], [], [], [], []
            jstart = jnp.int32(N)
            jend = jnp.int32(0)
            bmin = jnp.int32(127)
            bmax = jnp.int32(-1)
            for g in range(CPG):
                qidx = qbase + (sc * CPG + g) * L + lane
                qx.append(plsc.load_gather(xs_v, [qidx]))
                qy.append(plsc.load_gather(ys_v, [qidx]))
                qz.append(plsc.load_gather(zs_v, [qidx]))
                bq.append(plsc.load_gather(b_v, [qidx]))
                qn = qx[g] * qx[g] + qy[g] * qy[g] + qz[g] * qz[g]
                ht1.append((T1 - qn) * np.float32(0.5))
                ht2.append((T2 - qn) * np.float32(0.5))
                sv = plsc.load_gather(off_v, [bq[g]])
                ev = plsc.load_gather(off_v, [bq[g] + 1])
                jstart = jnp.minimum(jstart, jnp.min(sv))
                jend = jnp.maximum(jend, jnp.max(ev))
                bmin = jnp.minimum(bmin, jnp.min(bq[g]))
                bmax = jnp.maximum(bmax, jnp.max(bq[g]))

            both = jnp.int32(0x10001)
            one_i = jnp.int32(1)
            zero_i = jnp.int32(0)
            z16 = jnp.zeros((L,), jnp.int32)

            def make_body(lane_masks):
                def body(j, carry):
                    jv = jnp.full((L,), j, dtype=jnp.int32)
                    xj = plsc.load_gather(xs_v, [jv])
                    yj = plsc.load_gather(ys_v, [jv])
                    zj = plsc.load_gather(zs_v, [jv])
                    hwj = plsc.load_gather(hw_v, [jv])
                    out = []
                    for g in range(CPG):
                        u = hwj - (qx[g] * xj + qy[g] * yj + qz[g] * zj)
                        m1 = u <= ht1[g]
                        m2 = u <= ht2[g]
                        if lane_masks is not None:
                            m2 = m2 & lane_masks[g]
                        step = jnp.where(m2, jnp.where(m1, both, one_i), zero_i)
                        out.append(carry[g] + step)
                    return tuple(out)
                return body

            def run_fast(_):
                return plsc.parallel_loop(
                    jstart, jend, 1, unroll=2,
                    carry=(z16,) * CPG)(make_body(None))

            def run_spanning(_):
                def seg_body(s, carry):
                    sv = jnp.full((L,), s, dtype=jnp.int32)
                    lo = jnp.min(plsc.load_gather(off_v, [sv]))
                    hi = jnp.max(plsc.load_gather(off_v, [sv + 1]))
                    masks = [bq[g] == sv for g in range(CPG)]
                    return plsc.parallel_loop(
                        lo, hi, 1, unroll=2, carry=carry)(make_body(masks))
                return lax.fori_loop(bmin, bmax + 1, seg_body, (z16,) * CPG)

            acc = lax.cond(bmin == bmax, run_fast, run_spanning, 0)
            for g in range(CPG):
                o = (sc * CPG + g) * L
                c1 = (acc[g] >> 16).astype(jnp.float32)
                c2 = (acc[g] & jnp.int32(0xFFFF)).astype(jnp.float32)
                c1_v[pl.ds(o, L)] = (
                    jnp.minimum(c1, np.float32(32.0)) * np.float32(1.0 / 32.0))
                c2_v[pl.ds(o, L)] = (
                    jnp.minimum(c2, np.float32(64.0)) * np.float32(1.0 / 64.0))

        pltpu.sync_copy(c1_v, c1_h.at[pl.ds(qbase, QPW)])
        pltpu.sync_copy(c2_v, c2_h.at[pl.ds(qbase, QPW)])

    return k(xs, ys, zs, b32, off16)


def kernel(x, pos, batch):
    pos = pos.astype(jnp.float32)
    xs = pos[:, 0]
    ys = pos[:, 1]
    zs = pos[:, 2]
    b32 = batch.astype(jnp.int32)
    off16 = jnp.sum(b32[:, None] < jnp.arange(L, dtype=jnp.int32)[None, :],
                    axis=0, dtype=jnp.int32)
    c1, c2 = _sc_counts(xs, ys, zs, b32, off16)
    feats = jnp.concatenate([x, pos, c1[:, None], c2[:, None]], axis=1)
    return feats, pos, batch
